# baseline (speedup 1.0000x reference)
import jax
import jax.numpy as jnp
from jax.experimental import pallas as pl
from jax.experimental.pallas import tpu as pltpu

_IDX = 1
_M, _K = 8192, 2048
_R = 1024
_NCH = _M // _R
_IDEPTH = 4
_ODEPTH = 3


def _body(a_hbm, o_hbm, ibufs, obufs, isems, osems):
    irow = _IDX * _M

    def start_in(c):
        b = c % _IDEPTH
        return pltpu.async_copy(
            a_hbm.at[pl.ds(irow + c * _R, _R)], ibufs.at[b], isems.at[b]
        )

    def start_out(c):
        b = c % _ODEPTH
        return pltpu.async_copy(
            obufs.at[b], o_hbm.at[pl.ds(c * _R, _R)], osems.at[b]
        )

    hin = [None] * _NCH
    hout = [None] * _NCH
    for c in range(min(_IDEPTH, _NCH)):
        hin[c] = start_in(c)
    for c in range(_NCH):
        bi = c % _IDEPTH
        bo = c % _ODEPTH
        hin[c].wait()
        if c >= _ODEPTH:
            hout[c - _ODEPTH].wait()
        obufs[bo] = ibufs[bi] + ibufs[bi]
        hout[c] = start_out(c)
        if c + _IDEPTH < _NCH:
            hin[c + _IDEPTH] = start_in(c + _IDEPTH)
    for c in range(_NCH - _ODEPTH, _NCH):
        hout[c].wait()


def kernel(a):
    n, m, k = a.shape
    a2 = a.reshape(n * m, k)
    return pl.pallas_call(
        _body,
        in_specs=[pl.BlockSpec(memory_space=pltpu.HBM)],
        out_specs=pl.BlockSpec(memory_space=pltpu.HBM),
        out_shape=jax.ShapeDtypeStruct((m, k), a.dtype),
        scratch_shapes=[
            pltpu.VMEM((_IDEPTH, _R, _K), jnp.float32),
            pltpu.VMEM((_ODEPTH, _R, _K), jnp.float32),
            pltpu.SemaphoreType.DMA((_IDEPTH,)),
            pltpu.SemaphoreType.DMA((_ODEPTH,)),
        ],
    )(a2)


# speedup vs baseline: 1.0016x; 1.0016x over previous
import jax
import jax.numpy as jnp
from jax.experimental import pallas as pl
from jax.experimental.pallas import tpu as pltpu

_IDX = 1
_M, _K = 8192, 2048
_R = 1024
_NCH = _M // _R
_IDEPTH = 3
_ODEPTH = 3


def _body(a_hbm, o_hbm, ibufs, obufs, isems, osems):
    irow = _IDX * _M

    def start_in(c):
        b = c % _IDEPTH
        return pltpu.async_copy(
            a_hbm.at[pl.ds(irow + c * _R, _R)], ibufs.at[b], isems.at[b]
        )

    def start_out(c):
        b = c % _ODEPTH
        return pltpu.async_copy(
            obufs.at[b], o_hbm.at[pl.ds(c * _R, _R)], osems.at[b]
        )

    hin = [None] * _NCH
    hout = [None] * _NCH
    for c in range(min(_IDEPTH, _NCH)):
        hin[c] = start_in(c)
    for c in range(_NCH):
        bi = c % _IDEPTH
        bo = c % _ODEPTH
        hin[c].wait()
        if c >= _ODEPTH:
            hout[c - _ODEPTH].wait()
        obufs[bo] = ibufs[bi] + ibufs[bi]
        hout[c] = start_out(c)
        if c + _IDEPTH < _NCH:
            hin[c + _IDEPTH] = start_in(c + _IDEPTH)
    for c in range(_NCH - _ODEPTH, _NCH):
        hout[c].wait()


def kernel(a):
    n, m, k = a.shape
    a2 = a.reshape(n * m, k)
    return pl.pallas_call(
        _body,
        in_specs=[pl.BlockSpec(memory_space=pltpu.HBM)],
        out_specs=pl.BlockSpec(memory_space=pltpu.HBM),
        out_shape=jax.ShapeDtypeStruct((m, k), a.dtype),
        scratch_shapes=[
            pltpu.VMEM((_IDEPTH, _R, _K), jnp.float32),
            pltpu.VMEM((_ODEPTH, _R, _K), jnp.float32),
            pltpu.SemaphoreType.DMA((_IDEPTH,)),
            pltpu.SemaphoreType.DMA((_ODEPTH,)),
        ],
    )(a2)


# speedup vs baseline: 1.0167x; 1.0151x over previous
import jax
import jax.numpy as jnp
from jax.experimental import pallas as pl
from jax.experimental.pallas import tpu as pltpu

_IDX = 1
_M, _K = 8192, 2048
_RMAX = 1024
_CHUNKS = (256, 256, 512, 1024, 1024, 1024, 1024, 1024, 1024, 512, 256, 256)
assert sum(_CHUNKS) == _M
_NCH = len(_CHUNKS)
_OFFS = tuple(sum(_CHUNKS[:i]) for i in range(_NCH))
_DEPTH = 3


def _body(a_hbm, o_hbm, ibufs, obufs, isems, osems):
    irow = _IDX * _M

    def start_in(c):
        b = c % _DEPTH
        r = _CHUNKS[c]
        return pltpu.async_copy(
            a_hbm.at[pl.ds(irow + _OFFS[c], r)],
            ibufs.at[b, pl.ds(0, r)],
            isems.at[b],
        )

    def start_out(c):
        b = c % _DEPTH
        r = _CHUNKS[c]
        return pltpu.async_copy(
            obufs.at[b, pl.ds(0, r)],
            o_hbm.at[pl.ds(_OFFS[c], r)],
            osems.at[b],
        )

    hin = [None] * _NCH
    hout = [None] * _NCH
    for c in range(_DEPTH):
        hin[c] = start_in(c)
    for c in range(_NCH):
        b = c % _DEPTH
        r = _CHUNKS[c]
        hin[c].wait()
        if c >= _DEPTH:
            hout[c - _DEPTH].wait()
        obufs[b, pl.ds(0, r)] = ibufs[b, pl.ds(0, r)] + ibufs[b, pl.ds(0, r)]
        hout[c] = start_out(c)
        if c + _DEPTH < _NCH:
            hin[c + _DEPTH] = start_in(c + _DEPTH)
    for c in range(_NCH - _DEPTH, _NCH):
        hout[c].wait()


def kernel(a):
    n, m, k = a.shape
    a2 = a.reshape(n * m, k)
    return pl.pallas_call(
        _body,
        in_specs=[pl.BlockSpec(memory_space=pltpu.HBM)],
        out_specs=pl.BlockSpec(memory_space=pltpu.HBM),
        out_shape=jax.ShapeDtypeStruct((m, k), a.dtype),
        scratch_shapes=[
            pltpu.VMEM((_DEPTH, _RMAX, _K), jnp.float32),
            pltpu.VMEM((_DEPTH, _RMAX, _K), jnp.float32),
            pltpu.SemaphoreType.DMA((_DEPTH,)),
            pltpu.SemaphoreType.DMA((_DEPTH,)),
        ],
    )(a2)
